# baseline (device time: 31101 ns/iter reference)
import jax
import jax.numpy as jnp
from jax import lax
from jax.experimental import pallas as pl
from jax.experimental.pallas import tpu as pltpu

N_DEV = 16
N_TILES = 8


def kernel(x, w_mat, scale_x, scale_w):
    m_per, k = x.shape
    _, n = w_mat.shape
    n_per = n // N_DEV
    tile_n = n // N_TILES

    def body(x_ref, w_ref, sx_ref, sw_ref, out_ref,
             xb_ref, wtile_ref, wb_ref, sendbuf_ref, recvbuf_ref,
             copy_sems, send_sems, recv_sems):
        my = lax.axis_index("i")

        barrier = pltpu.get_barrier_semaphore()
        for off in range(1, N_DEV):
            pl.semaphore_signal(
                barrier, inc=1,
                device_id=((my + off) % N_DEV,),
                device_id_type=pl.DeviceIdType.MESH,
            )

        t0 = my // 2

        def tile_col(s):
            return ((t0 + s) % N_TILES) * tile_n

        first = pltpu.make_async_copy(
            w_ref.at[:, pl.ds(tile_col(0), tile_n)],
            wtile_ref.at[0], copy_sems.at[0])
        first.start()
        xb_ref[...] = x_ref[...].astype(jnp.bfloat16)
        s_scale = sx_ref[0] * sw_ref[0]

        pl.semaphore_wait(barrier, N_DEV - 1)

        for s in range(N_TILES):
            slot = s % 2
            if s + 1 < N_TILES:
                nxt = pltpu.make_async_copy(
                    w_ref.at[:, pl.ds(tile_col(s + 1), tile_n)],
                    wtile_ref.at[(s + 1) % 2], copy_sems.at[(s + 1) % 2])
                nxt.start()
            pltpu.make_async_copy(
                w_ref.at[:, pl.ds(tile_col(s), tile_n)],
                wtile_ref.at[slot], copy_sems.at[slot]).wait()

            wb_ref[...] = wtile_ref[slot].astype(jnp.bfloat16)
            acc = jnp.dot(xb_ref[...], wb_ref[...],
                          preferred_element_type=jnp.float32)
            blk = jnp.maximum(acc * s_scale, 0.0)

            for half in range(2):
                b = (t0 + s) % N_TILES * 2 + half
                sub = blk[:, half * n_per:(half + 1) * n_per]

                @pl.when(b == my)
                def _(sub=sub):
                    out_ref[pl.ds(my * m_per, m_per), :] = sub

                @pl.when(b != my)
                def _(sub=sub, b=b):
                    sendbuf_ref[b] = sub.astype(jnp.bfloat16)
                    rdma = pltpu.make_async_remote_copy(
                        src_ref=sendbuf_ref.at[b],
                        dst_ref=recvbuf_ref.at[pl.ds(my * m_per, m_per), :],
                        send_sem=send_sems.at[b],
                        recv_sem=recv_sems.at[my],
                        device_id=(b,),
                        device_id_type=pl.DeviceIdType.MESH,
                    )
                    rdma.start()

        for off in range(1, N_DEV):
            src = (my + off) % N_DEV
            recv = pltpu.make_async_remote_copy(
                src_ref=sendbuf_ref.at[src],
                dst_ref=recvbuf_ref.at[pl.ds(src * m_per, m_per), :],
                send_sem=send_sems.at[src],
                recv_sem=recv_sems.at[src],
                device_id=(src,),
                device_id_type=pl.DeviceIdType.MESH,
            )
            recv.wait_recv()
            out_ref[pl.ds(src * m_per, m_per), :] = (
                recvbuf_ref[pl.ds(src * m_per, m_per), :].astype(jnp.float32))

        for off in range(1, N_DEV):
            dst = (my + off) % N_DEV
            send = pltpu.make_async_remote_copy(
                src_ref=sendbuf_ref.at[dst],
                dst_ref=recvbuf_ref.at[pl.ds(my * m_per, m_per), :],
                send_sem=send_sems.at[dst],
                recv_sem=recv_sems.at[my],
                device_id=(dst,),
                device_id_type=pl.DeviceIdType.MESH,
            )
            send.wait_send()

    out_shape = jax.ShapeDtypeStruct((N_DEV * m_per, n_per), jnp.float32)
    return pl.pallas_call(
        body,
        out_shape=out_shape,
        in_specs=[
            pl.BlockSpec(memory_space=pltpu.VMEM),
            pl.BlockSpec(memory_space=pltpu.MemorySpace.HBM),
            pl.BlockSpec(memory_space=pltpu.SMEM),
            pl.BlockSpec(memory_space=pltpu.SMEM),
        ],
        out_specs=pl.BlockSpec(memory_space=pltpu.VMEM),
        scratch_shapes=[
            pltpu.VMEM((m_per, k), jnp.bfloat16),
            pltpu.VMEM((2, k, tile_n), jnp.float32),
            pltpu.VMEM((k, tile_n), jnp.bfloat16),
            pltpu.VMEM((N_DEV, m_per, n_per), jnp.bfloat16),
            pltpu.VMEM((N_DEV * m_per, n_per), jnp.bfloat16),
            pltpu.SemaphoreType.DMA((2,)),
            pltpu.SemaphoreType.DMA((N_DEV,)),
            pltpu.SemaphoreType.DMA((N_DEV,)),
        ],
        compiler_params=pltpu.CompilerParams(
            collective_id=0, vmem_limit_bytes=100 * 1024 * 1024),
    )(x, w_mat, scale_x, scale_w)


# device time: 27373 ns/iter; 1.1362x vs baseline; 1.1362x over previous
import jax
import jax.numpy as jnp
from jax import lax
from jax.experimental import pallas as pl
from jax.experimental.pallas import tpu as pltpu

import os

N_DEV = 16
N_TILES = int(os.environ.get("KERNEL_NTILES", "16"))
WINDOW = min(int(os.environ.get("KERNEL_WINDOW", "8")), N_TILES)
_COMM = os.environ.get("KERNEL_NO_COMM", "0") != "1"
GROUP = int(os.environ.get("KERNEL_GROUP", "1"))
_PROBE = os.environ.get("KERNEL_PROBE", "")
_OPD = (jnp.float8_e4m3fn if os.environ.get("KERNEL_FP8", "1") == "1"
        else jnp.bfloat16)


def kernel(x, w_mat, scale_x, scale_w):
    m_per, k = x.shape
    _, n = w_mat.shape
    n_per = n // N_DEV
    tile_n = n // N_TILES

    def body(x_ref, w_ref, sx_ref, sw_ref, out_ref,
             xb_ref, wtile_ref, wb_ref, sendbuf_ref, recvbuf_ref,
             copy_sems, send_sems, recv_sems):
        my = lax.axis_index("i")

        barrier = pltpu.get_barrier_semaphore()
        for off in range(1, N_DEV):
            pl.semaphore_signal(
                barrier, inc=1,
                device_id=((my + off) % N_DEV,),
                device_id_type=pl.DeviceIdType.MESH,
            )

        bpt = N_DEV // N_TILES
        t0 = my // bpt

        PERM = [12, 13, 14, 15, 8, 9, 10, 11, 4, 5, 6, 7, 1, 2, 3, 0]
        PERM8 = [6, 7, 4, 5, 2, 3, 1, 0]

        if _PROBE == "rowstream":
            rows = k // N_TILES
            pltpu.make_async_copy(
                w_ref.at[pl.ds(0, rows), :], wtile_ref.at[0],
                copy_sems.at[0]).start()
            for s in range(N_TILES):
                slot = s % 2
                if s + 1 < N_TILES:
                    pltpu.make_async_copy(
                        w_ref.at[pl.ds((s + 1) * rows, rows), :],
                        wtile_ref.at[(s + 1) % 2],
                        copy_sems.at[(s + 1) % 2]).start()
                pltpu.make_async_copy(
                    w_ref.at[pl.ds(s * rows, rows), :],
                    wtile_ref.at[slot], copy_sems.at[slot]).wait()
            return

        def tile_idx(s):
            if bpt == 1:
                return (my + PERM[s]) % N_DEV
            if bpt == 2:
                return (t0 + PERM8[s]) % N_TILES
            return (t0 + s) % N_TILES

        def tile_col(s):
            return tile_idx(s) * tile_n

        pltpu.make_async_copy(
            w_ref.at[:, pl.ds(tile_col(0), tile_n)],
            wtile_ref.at[0], copy_sems.at[0]).start()
        xb_ref[...] = x_ref[...].astype(_OPD)
        for s in range(1, WINDOW):
            pltpu.make_async_copy(
                w_ref.at[:, pl.ds(tile_col(s), tile_n)],
                wtile_ref.at[s % WINDOW], copy_sems.at[s % WINDOW]).start()
        s_scale = sx_ref[0] * sw_ref[0]

        pl.semaphore_wait(barrier, N_DEV - 1)

        for g in range(N_TILES // GROUP):
            for c in range(GROUP):
                s = g * GROUP + c
                slot = s % WINDOW
                pltpu.make_async_copy(
                    w_ref.at[:, pl.ds(tile_col(s), tile_n)],
                    wtile_ref.at[slot], copy_sems.at[slot]).wait()

                if _PROBE != "nowork":
                    wb_ref[:, pl.ds(c * tile_n, tile_n)] = (
                        wtile_ref[slot].astype(_OPD))
                if s + WINDOW < N_TILES:
                    pltpu.make_async_copy(
                        w_ref.at[:, pl.ds(tile_col(s + WINDOW), tile_n)],
                        wtile_ref.at[slot], copy_sems.at[slot]).start()
            if _PROBE in ("nowork", "nodot"):
                continue
            acc = jnp.dot(xb_ref[...], wb_ref[...],
                          preferred_element_type=jnp.float32)
            blk = jnp.maximum(acc * s_scale, 0.0)

            for c in range(GROUP):
                s = g * GROUP + c
                for half in range(bpt):
                    b = tile_idx(s) * bpt + half
                    sub = blk[:, (c * bpt + half) * n_per:
                              (c * bpt + half + 1) * n_per]

                    @pl.when(b == my)
                    def _(sub=sub):
                        out_ref[pl.ds(my * m_per, m_per), :] = sub

                    @pl.when(b != my)
                    def _(sub=sub, b=b):
                        sendbuf_ref[b] = sub.astype(jnp.bfloat16)
                        if not _COMM:
                            return
                        rdma = pltpu.make_async_remote_copy(
                            src_ref=sendbuf_ref.at[b],
                            dst_ref=recvbuf_ref.at[
                                pl.ds(my * m_per, m_per), :],
                            send_sem=send_sems.at[b],
                            recv_sem=recv_sems.at[my],
                            device_id=(b,),
                            device_id_type=pl.DeviceIdType.MESH,
                        )
                        rdma.start()

        for off in range(1, N_DEV if _COMM else 0):
            src = (my + off) % N_DEV
            recv = pltpu.make_async_remote_copy(
                src_ref=sendbuf_ref.at[src],
                dst_ref=recvbuf_ref.at[pl.ds(src * m_per, m_per), :],
                send_sem=send_sems.at[src],
                recv_sem=recv_sems.at[src],
                device_id=(src,),
                device_id_type=pl.DeviceIdType.MESH,
            )
            recv.wait_recv()
            out_ref[pl.ds(src * m_per, m_per), :] = (
                recvbuf_ref[pl.ds(src * m_per, m_per), :].astype(jnp.float32))

        for off in range(1, N_DEV if _COMM else 0):
            dst = (my + off) % N_DEV
            send = pltpu.make_async_remote_copy(
                src_ref=sendbuf_ref.at[dst],
                dst_ref=recvbuf_ref.at[pl.ds(my * m_per, m_per), :],
                send_sem=send_sems.at[dst],
                recv_sem=recv_sems.at[my],
                device_id=(dst,),
                device_id_type=pl.DeviceIdType.MESH,
            )
            send.wait_send()

    out_shape = jax.ShapeDtypeStruct((N_DEV * m_per, n_per), jnp.float32)
    return pl.pallas_call(
        body,
        out_shape=out_shape,
        in_specs=[
            pl.BlockSpec(memory_space=pltpu.VMEM),
            pl.BlockSpec(memory_space=pltpu.MemorySpace.HBM),
            pl.BlockSpec(memory_space=pltpu.SMEM),
            pl.BlockSpec(memory_space=pltpu.SMEM),
        ],
        out_specs=pl.BlockSpec(memory_space=pltpu.VMEM),
        scratch_shapes=[
            pltpu.VMEM((m_per, k), _OPD),
            (pltpu.VMEM((2, k // N_TILES, n), jnp.float32)
             if _PROBE == "rowstream" else
             pltpu.VMEM((WINDOW, k, tile_n), jnp.float32)),
            pltpu.VMEM((k, GROUP * tile_n), _OPD),
            pltpu.VMEM((N_DEV, m_per, n_per), jnp.bfloat16),
            pltpu.VMEM((N_DEV * m_per, n_per), jnp.bfloat16),
            pltpu.SemaphoreType.DMA((WINDOW,)),
            pltpu.SemaphoreType.DMA((N_DEV,)),
            pltpu.SemaphoreType.DMA((N_DEV,)),
        ],
        compiler_params=pltpu.CompilerParams(
            collective_id=0, vmem_limit_bytes=100 * 1024 * 1024),
    )(x, w_mat, scale_x, scale_w)


# device time: 27265 ns/iter; 1.1407x vs baseline; 1.0040x over previous
import jax
import jax.numpy as jnp
from jax import lax
from jax.experimental import pallas as pl
from jax.experimental.pallas import tpu as pltpu

import os

N_DEV = 16
N_TILES = int(os.environ.get("KERNEL_NTILES", "16"))
WINDOW = min(int(os.environ.get("KERNEL_WINDOW", "8")), N_TILES)
_COMM = os.environ.get("KERNEL_NO_COMM", "0") != "1"
GROUP = int(os.environ.get("KERNEL_GROUP", "1"))
_PROBE = os.environ.get("KERNEL_PROBE", "")
_OPD = (jnp.float8_e4m3fn if os.environ.get("KERNEL_FP8", "1") == "1"
        else jnp.bfloat16)


def kernel(x, w_mat, scale_x, scale_w):
    m_per, k = x.shape
    _, n = w_mat.shape
    n_per = n // N_DEV
    tile_n = n // N_TILES

    def body(x_ref, w_ref, sx_ref, sw_ref, out_ref,
             xb_ref, wtile_ref, wb_ref, sendbuf_ref, recvbuf_ref,
             stage_ref, copy_sems, send_sems, recv_sems, stage_sems):
        my = lax.axis_index("i")

        barrier = pltpu.get_barrier_semaphore()
        for off in range(1, N_DEV):
            pl.semaphore_signal(
                barrier, inc=1,
                device_id=((my + off) % N_DEV,),
                device_id_type=pl.DeviceIdType.MESH,
            )

        bpt = N_DEV // N_TILES
        t0 = my // bpt

        PERM = [12, 13, 14, 15, 8, 9, 10, 11, 4, 5, 6, 7, 1, 2, 3, 0]
        PERM8 = [6, 7, 4, 5, 2, 3, 1, 0]

        if _PROBE == "rowstream":
            rows = k // N_TILES
            pltpu.make_async_copy(
                w_ref.at[pl.ds(0, rows), :], wtile_ref.at[0],
                copy_sems.at[0]).start()
            for s in range(N_TILES):
                slot = s % 2
                if s + 1 < N_TILES:
                    pltpu.make_async_copy(
                        w_ref.at[pl.ds((s + 1) * rows, rows), :],
                        wtile_ref.at[(s + 1) % 2],
                        copy_sems.at[(s + 1) % 2]).start()
                pltpu.make_async_copy(
                    w_ref.at[pl.ds(s * rows, rows), :],
                    wtile_ref.at[slot], copy_sems.at[slot]).wait()
            return

        def tile_idx(s):
            if bpt == 1:
                return (my + PERM[s]) % N_DEV
            if bpt == 2:
                return (t0 + PERM8[s]) % N_TILES
            return (t0 + s) % N_TILES

        def tile_col(s):
            return tile_idx(s) * tile_n

        pltpu.make_async_copy(
            w_ref.at[:, pl.ds(tile_col(0), tile_n)],
            wtile_ref.at[0], copy_sems.at[0]).start()
        xb_ref[...] = x_ref[...].astype(_OPD)
        for s in range(1, WINDOW):
            pltpu.make_async_copy(
                w_ref.at[:, pl.ds(tile_col(s), tile_n)],
                wtile_ref.at[s % WINDOW], copy_sems.at[s % WINDOW]).start()
        s_scale = sx_ref[0] * sw_ref[0]

        pl.semaphore_wait(barrier, N_DEV - 1)

        for g in range(N_TILES // GROUP):
            for c in range(GROUP):
                s = g * GROUP + c
                slot = s % WINDOW
                pltpu.make_async_copy(
                    w_ref.at[:, pl.ds(tile_col(s), tile_n)],
                    wtile_ref.at[slot], copy_sems.at[slot]).wait()

                if _PROBE != "nowork":
                    wb_ref[:, pl.ds(c * tile_n, tile_n)] = (
                        wtile_ref[slot].astype(_OPD))
                if s + WINDOW < N_TILES:
                    pltpu.make_async_copy(
                        w_ref.at[:, pl.ds(tile_col(s + WINDOW), tile_n)],
                        wtile_ref.at[slot], copy_sems.at[slot]).start()
            if _PROBE in ("nowork", "nodot"):
                continue
            acc = jnp.dot(xb_ref[...], wb_ref[...],
                          preferred_element_type=jnp.float32)
            blk = jnp.maximum(acc * s_scale, 0.0)

            for c in range(GROUP):
                s = g * GROUP + c
                for half in range(bpt):
                    b = tile_idx(s) * bpt + half
                    sub = blk[:, (c * bpt + half) * n_per:
                              (c * bpt + half + 1) * n_per]

                    @pl.when(b == my)
                    def _(sub=sub):
                        stage_ref[0] = sub
                        pltpu.make_async_copy(
                            stage_ref.at[0],
                            out_ref.at[pl.ds(my * m_per, m_per), :],
                            stage_sems.at[0]).start()

                    @pl.when(b != my)
                    def _(sub=sub, b=b):
                        sendbuf_ref[b] = sub.astype(jnp.bfloat16)
                        if not _COMM:
                            return
                        rdma = pltpu.make_async_remote_copy(
                            src_ref=sendbuf_ref.at[b],
                            dst_ref=recvbuf_ref.at[
                                pl.ds(my * m_per, m_per), :],
                            send_sem=send_sems.at[b],
                            recv_sem=recv_sems.at[my],
                            device_id=(b,),
                            device_id_type=pl.DeviceIdType.MESH,
                        )
                        rdma.start()

        for off in range(1, N_DEV if _COMM else 0):
            src = (my + off) % N_DEV
            recv = pltpu.make_async_remote_copy(
                src_ref=sendbuf_ref.at[src],
                dst_ref=recvbuf_ref.at[pl.ds(src * m_per, m_per), :],
                send_sem=send_sems.at[src],
                recv_sem=recv_sems.at[src],
                device_id=(src,),
                device_id_type=pl.DeviceIdType.MESH,
            )
            recv.wait_recv()
            slot = 1 + (off - 1) % 3
            if off > 3:
                pltpu.make_async_copy(
                    stage_ref.at[slot],
                    out_ref.at[pl.ds(src * m_per, m_per), :],
                    stage_sems.at[slot]).wait()
            stage_ref[slot] = (
                recvbuf_ref[pl.ds(src * m_per, m_per), :].astype(jnp.float32))
            pltpu.make_async_copy(
                stage_ref.at[slot],
                out_ref.at[pl.ds(src * m_per, m_per), :],
                stage_sems.at[slot]).start()

        if _COMM:
            for slot in range(4):
                pltpu.make_async_copy(
                    stage_ref.at[slot],
                    out_ref.at[pl.ds(0, m_per), :],
                    stage_sems.at[slot]).wait()

        for off in range(1, N_DEV if _COMM else 0):
            dst = (my + off) % N_DEV
            send = pltpu.make_async_remote_copy(
                src_ref=sendbuf_ref.at[dst],
                dst_ref=recvbuf_ref.at[pl.ds(my * m_per, m_per), :],
                send_sem=send_sems.at[dst],
                recv_sem=recv_sems.at[my],
                device_id=(dst,),
                device_id_type=pl.DeviceIdType.MESH,
            )
            send.wait_send()

    out_shape = jax.ShapeDtypeStruct((N_DEV * m_per, n_per), jnp.float32)
    return pl.pallas_call(
        body,
        out_shape=out_shape,
        in_specs=[
            pl.BlockSpec(memory_space=pltpu.VMEM),
            pl.BlockSpec(memory_space=pltpu.MemorySpace.HBM),
            pl.BlockSpec(memory_space=pltpu.SMEM),
            pl.BlockSpec(memory_space=pltpu.SMEM),
        ],
        out_specs=pl.BlockSpec(memory_space=pltpu.MemorySpace.HBM),
        scratch_shapes=[
            pltpu.VMEM((m_per, k), _OPD),
            (pltpu.VMEM((2, k // N_TILES, n), jnp.float32)
             if _PROBE == "rowstream" else
             pltpu.VMEM((WINDOW, k, tile_n), jnp.float32)),
            pltpu.VMEM((k, GROUP * tile_n), _OPD),
            pltpu.VMEM((N_DEV, m_per, n_per), jnp.bfloat16),
            pltpu.VMEM((N_DEV * m_per, n_per), jnp.bfloat16),
            pltpu.VMEM((4, m_per, n_per), jnp.float32),
            pltpu.SemaphoreType.DMA((WINDOW,)),
            pltpu.SemaphoreType.DMA((N_DEV,)),
            pltpu.SemaphoreType.DMA((N_DEV,)),
            pltpu.SemaphoreType.DMA((4,)),
        ],
        compiler_params=pltpu.CompilerParams(
            collective_id=0, vmem_limit_bytes=100 * 1024 * 1024),
    )(x, w_mat, scale_x, scale_w)
